# Initial kernel scaffold; baseline (speedup 1.0000x reference)
import jax
import jax.numpy as jnp
from jax.experimental import pallas as pl


def kernel(tickers, table):
    raise NotImplementedError("write your pallas kernel here")


# speedup vs baseline: 2.2682x; 2.2682x over previous
import functools

import jax
import jax.numpy as jnp
from jax import lax
from jax.experimental import pallas as pl
from jax.experimental.pallas import tpu as pltpu
from jax.experimental.pallas import tpu_sc as plsc

_NUM_CORES = 2
_NUM_SUBCORES = 16
_NUM_WORKERS = _NUM_CORES * _NUM_SUBCORES
_CHUNK = 128


@functools.cache
def _build(batch, vocab, dim):
  b_per_w = batch // _NUM_WORKERS
  n_chunks = b_per_w // _CHUNK
  mesh = plsc.VectorSubcoreMesh(core_axis_name="c", subcore_axis_name="s")

  @functools.partial(
      pl.kernel,
      mesh=mesh,
      out_type=jax.ShapeDtypeStruct((batch, dim), jnp.float32),
      scratch_types=[
          pltpu.VMEM((n_chunks, _CHUNK), jnp.int32),
          pltpu.VMEM((b_per_w, dim), jnp.float32),
          pltpu.SemaphoreType.DMA,
      ],
      compiler_params=pltpu.CompilerParams(use_tc_tiling_on_sc=False),
  )
  def emb(tickers_hbm, table_hbm, out_hbm, idx_v, rows_v, sem):
    wid = lax.axis_index("s") * _NUM_CORES + lax.axis_index("c")
    base = wid * b_per_w
    pltpu.sync_copy(tickers_hbm.at[wid], idx_v)
    copies = [
        pltpu.async_copy(
            table_hbm.at[idx_v.at[j]],
            rows_v.at[pl.ds(j * _CHUNK, _CHUNK)],
            sem,
        )
        for j in range(n_chunks)
    ]
    for c in copies:
      c.wait()
    pltpu.sync_copy(rows_v, out_hbm.at[pl.ds(base, b_per_w)])

  return emb


def kernel(tickers, table):
  batch = tickers.shape[0]
  vocab, dim = table.shape
  b_per_w = batch // _NUM_WORKERS
  idx = tickers.reshape(_NUM_WORKERS, b_per_w // _CHUNK, _CHUNK)
  return _build(batch, vocab, dim)(idx, table)


# speedup vs baseline: 2.6909x; 1.1864x over previous
import functools

import jax
import jax.numpy as jnp
from jax import lax
from jax.experimental import pallas as pl
from jax.experimental.pallas import tpu as pltpu
from jax.experimental.pallas import tpu_sc as plsc

_NUM_CORES = 2
_NUM_SUBCORES = 16
_NUM_WORKERS = _NUM_CORES * _NUM_SUBCORES
_CHUNK = 128
_LANES = 16


@functools.cache
def _build(batch, vocab, dim):
  b_per_w = batch // _NUM_WORKERS
  n_chunks = b_per_w // _CHUNK
  mesh = plsc.VectorSubcoreMesh(core_axis_name="c", subcore_axis_name="s")

  @functools.partial(
      pl.kernel,
      mesh=mesh,
      out_type=jax.ShapeDtypeStruct((dim, batch), jnp.float32),
      scratch_types=[
          pltpu.VMEM((b_per_w,), jnp.int32),
          pltpu.VMEM((b_per_w, dim), jnp.float32),
          pltpu.VMEM((dim, b_per_w), jnp.float32),
          pltpu.SemaphoreType.DMA,
      ],
      compiler_params=pltpu.CompilerParams(
          use_tc_tiling_on_sc=False, needs_layout_passes=False
      ),
  )
  def emb(tickers_hbm, table_hbm, out_hbm, idx_v, rows_v, tr_v, sem):
    wid = lax.axis_index("s") * _NUM_CORES + lax.axis_index("c")
    base = wid * b_per_w
    pltpu.sync_copy(tickers_hbm.at[pl.ds(base, b_per_w)], idx_v)
    copies = [
        pltpu.async_copy(
            table_hbm.at[idx_v.at[pl.ds(j * _CHUNK, _CHUNK)]],
            rows_v.at[pl.ds(j * _CHUNK, _CHUNK)],
            sem,
        )
        for j in range(n_chunks)
    ]
    for c in copies:
      c.wait()

    lanes = lax.iota(jnp.int32, _LANES)

    def transpose_group(k, carry):
      rowbase = k * _LANES
      ridx = rowbase + lanes
      for d in range(dim):
        vals = plsc.load_gather(rows_v, [ridx, jnp.full((_LANES,), d, jnp.int32)])
        tr_v[d, pl.ds(pl.multiple_of(rowbase, _LANES), _LANES)] = vals
      return carry

    lax.fori_loop(0, b_per_w // _LANES, transpose_group, 0)
    pltpu.sync_copy(tr_v, out_hbm.at[:, pl.ds(base, b_per_w)])

  return emb


def kernel(tickers, table):
  batch = tickers.shape[0]
  vocab, dim = table.shape
  out_t = _build(batch, vocab, dim)(tickers, table)
  return out_t.T


# speedup vs baseline: 3.0033x; 1.1161x over previous
import functools

import jax
import jax.numpy as jnp
from jax import lax
from jax.experimental import pallas as pl
from jax.experimental.pallas import tpu as pltpu
from jax.experimental.pallas import tpu_sc as plsc

_NUM_CORES = 2
_NUM_SUBCORES = 16
_NUM_WORKERS = _NUM_CORES * _NUM_SUBCORES
_LANES = 16
_SUB = 8
_LANE_T = 128


@functools.cache
def _build(batch, vocab, dim):
  b_per_w = batch // _NUM_WORKERS
  n_lane_t = b_per_w // _LANE_T
  n_sub_t = dim // _SUB
  mesh = plsc.VectorSubcoreMesh(core_axis_name="c", subcore_axis_name="s")

  @functools.partial(
      pl.kernel,
      mesh=mesh,
      out_type=jax.ShapeDtypeStruct(
          (n_sub_t, batch // _LANE_T, _SUB, _LANE_T), jnp.float32
      ),
      scratch_types=[
          pltpu.VMEM((b_per_w,), jnp.int32),
          pltpu.VMEM((dim, vocab), jnp.float32),
          pltpu.VMEM((n_sub_t, n_lane_t, _SUB, _LANE_T), jnp.float32),
          pltpu.SemaphoreType.DMA,
      ],
      compiler_params=pltpu.CompilerParams(
          use_tc_tiling_on_sc=False, needs_layout_passes=False
      ),
  )
  def emb(tickers_hbm, table_t_hbm, out_hbm, idx_v, tab_v, tr_v, sem):
    wid = lax.axis_index("s") * _NUM_CORES + lax.axis_index("c")
    base = wid * b_per_w
    c_idx = pltpu.async_copy(
        tickers_hbm.at[pl.ds(base, b_per_w)], idx_v, sem
    )
    c_tab = pltpu.async_copy(table_t_hbm, tab_v, sem)
    c_idx.wait()
    c_tab.wait()

    def lookup_tile(j, carry):
      for g8 in range(_LANE_T // _LANES):
        off = g8 * _LANES
        t16 = idx_v[pl.ds(j * _LANE_T + off, _LANES)]
        for d in range(dim):
          vals = plsc.load_gather(
              tab_v, [jnp.full((_LANES,), d, jnp.int32), t16]
          )
          tr_v[d // _SUB, j, d % _SUB, pl.ds(off, _LANES)] = vals
      return carry

    lax.fori_loop(0, n_lane_t, lookup_tile, 0)
    for r in range(n_sub_t):
      pltpu.sync_copy(
          tr_v.at[r],
          out_hbm.at[r, pl.ds(wid * n_lane_t, n_lane_t)],
      )

  return emb


def kernel(tickers, table):
  batch = tickers.shape[0]
  vocab, dim = table.shape
  oh = _build(batch, vocab, dim)(tickers, table.T)
  out_t = oh.transpose(0, 2, 1, 3).reshape(dim, batch)
  return out_t.T


# speedup vs baseline: 3.0890x; 1.0286x over previous
import functools

import jax
import jax.numpy as jnp
from jax import lax
from jax.experimental import pallas as pl
from jax.experimental.pallas import tpu as pltpu
from jax.experimental.pallas import tpu_sc as plsc

_NUM_CORES = 2
_NUM_SUBCORES = 16
_NUM_WORKERS = _NUM_CORES * _NUM_SUBCORES
_LANES = 16
_SUB = 8
_LANE_T = 128


@functools.cache
def _build(batch, vocab, dim):
  b_per_w = batch // _NUM_WORKERS
  n_lane_t = b_per_w // _LANE_T
  n_sub_t = dim // _SUB
  mesh = plsc.VectorSubcoreMesh(core_axis_name="c", subcore_axis_name="s")

  @functools.partial(
      pl.kernel,
      mesh=mesh,
      out_type=jax.ShapeDtypeStruct(
          (n_sub_t, batch // _LANE_T, _SUB, _LANE_T), jnp.float32
      ),
      scratch_types=[
          pltpu.VMEM((b_per_w,), jnp.int32),
          pltpu.VMEM((dim, vocab), jnp.float32),
          pltpu.VMEM((n_sub_t, n_lane_t, _SUB, _LANE_T), jnp.float32),
          pltpu.SemaphoreType.DMA,
      ],
      compiler_params=pltpu.CompilerParams(
          use_tc_tiling_on_sc=False, needs_layout_passes=False
      ),
  )
  def emb(tickers_hbm, table_t_hbm, out_hbm, idx_v, tab_v, tr_v, sem):
    wid = lax.axis_index("s") * _NUM_CORES + lax.axis_index("c")
    base = wid * b_per_w
    c_idx = pltpu.async_copy(
        tickers_hbm.at[pl.ds(base, b_per_w)], idx_v, sem
    )
    c_tab = pltpu.async_copy(table_t_hbm, tab_v, sem)
    c_idx.wait()
    c_tab.wait()

    def lookup_group(g, carry):
      j = g // (_LANE_T // _LANES)
      off = (g % (_LANE_T // _LANES)) * _LANES
      t16 = idx_v[pl.ds(g * _LANES, _LANES)]
      for d in range(dim):
        vals = plsc.load_gather(
            tab_v, [jnp.full((_LANES,), d, jnp.int32), t16]
        )
        tr_v[d // _SUB, j, d % _SUB, pl.ds(off, _LANES)] = vals
      return carry

    lax.fori_loop(0, b_per_w // _LANES, lookup_group, 0)
    for r in range(n_sub_t):
      pltpu.sync_copy(
          tr_v.at[r],
          out_hbm.at[r, pl.ds(wid * n_lane_t, n_lane_t)],
      )

  return emb


def kernel(tickers, table):
  batch = tickers.shape[0]
  vocab, dim = table.shape
  oh = _build(batch, vocab, dim)(tickers, table.T)
  out_t = oh.transpose(0, 2, 1, 3).reshape(dim, batch)
  return out_t.T
